# baseline (speedup 1.0000x reference)
import functools

import jax
import jax.numpy as jnp
from jax import lax
from jax.experimental import pallas as pl
from jax.experimental.pallas import tpu as pltpu
from jax.experimental.pallas import tpu_sc as plsc

_N = 10000
_E = 320000
_D = 128
_T = 3
_NC = 2
_NS = 16
_NW = _NC * _NS
_K = 128
_CHUNKS = _E // _K
_CPW = _CHUNKS // _NW
_CREM = _CHUNKS - _CPW * _NW
_NPAD = 10112
_RPT = _NPAD // _NS


_ABL_SCALE = True
_ABL_GATHER = True
_ABL_SCATTER = True


def _agg_body(src_hbm, dst_hbm, w_hbm, g_hbm, zero_hbm, out_hbm,
              sidx, didx, wv, rows0, acc, gsem, ssem):
    c = lax.axis_index("c")
    s = lax.axis_index("s")
    r0 = s * _RPT
    pltpu.sync_copy(zero_hbm.at[pl.ds(r0, _RPT)], acc.at[pl.ds(r0, _RPT)])
    plsc.subcore_barrier()
    wid = s * _NC + c
    base = wid * _CPW + jnp.minimum(wid, _CREM)
    count = _CPW + jnp.where(wid < _CREM, 1, 0)
    rows = rows0

    def chunk(g, carry):
        ch = base + g
        pltpu.sync_copy(src_hbm.at[ch], sidx)
        pltpu.sync_copy(dst_hbm.at[ch], didx)
        pltpu.sync_copy(w_hbm.at[ch], wv)
        if _ABL_GATHER:
            pltpu.async_copy(g_hbm.at[sidx], rows, gsem).wait()
        if _ABL_SCALE:
            def grp(jb, carry2):
                w16v = wv[pl.ds(jb * 16, 16)]
                for l in range(16):
                    w16 = lax.broadcast(w16v[l], (16,))
                    e = jb * 16 + l
                    for cb in range(8):
                        v = rows[e, pl.ds(cb * 16, 16)]
                        rows[e, pl.ds(cb * 16, 16)] = v * w16
                return carry2
            lax.fori_loop(0, _K // 16, grp, 0)
        if _ABL_SCATTER:
            pltpu.async_copy(rows, acc.at[didx], ssem, add=True).wait()
        return carry
    lax.fori_loop(0, _CPW, chunk, 0)
    plsc.subcore_barrier()
    pltpu.sync_copy(acc.at[pl.ds(r0, _RPT)], out_hbm.at[c, pl.ds(r0, _RPT)])


_sc_agg = functools.partial(
    pl.kernel,
    out_type=jax.ShapeDtypeStruct((_NC, _NPAD, _D), jnp.float32),
    mesh=plsc.VectorSubcoreMesh(core_axis_name="c", subcore_axis_name="s"),
    scratch_types=[
        pltpu.VMEM((_K,), jnp.int32),
        pltpu.VMEM((_K,), jnp.int32),
        pltpu.VMEM((_K,), jnp.float32),
        pltpu.VMEM((_K, _D), jnp.float32),
        pltpu.VMEM_SHARED((_NPAD, _D), jnp.float32),
        pltpu.SemaphoreType.DMA,
        pltpu.SemaphoreType.DMA,
    ],
)(_agg_body)


_BR = 1000


def _prep_body(degp_ref, x_ref, w0_ref, g_ref, dinvb_ref):
    d = degp_ref[0, 0] + degp_ref[0, 1]
    dinvb = lax.rsqrt(d + 1.0)
    hw = jnp.dot(x_ref[...], w0_ref[...], preferred_element_type=jnp.float32)
    g_ref[0] = hw * dinvb
    dinvb_ref[0] = dinvb


def _tc_prep(degp, x, w0):
    return pl.pallas_call(
        _prep_body,
        grid=(_T, _N // _BR),
        in_specs=[
            pl.BlockSpec((1, _NC, _BR, _D), lambda t, i: (t, 0, i, 0)),
            pl.BlockSpec((_BR, _D), lambda t, i: (i, 0)),
            pl.BlockSpec((_D, _D), lambda t, i: (0, 0)),
        ],
        out_specs=[
            pl.BlockSpec((1, _BR, _D), lambda t, i: (t, i, 0)),
            pl.BlockSpec((1, _BR, _D), lambda t, i: (t, i, 0)),
        ],
        out_shape=[
            jax.ShapeDtypeStruct((_T, _N, _D), jnp.float32),
            jax.ShapeDtypeStruct((_T, _N, _D), jnp.float32),
        ],
    )(degp, x, w0)


def _mid_body(sp_ref, g_ref, dinvb_ref, b_ref, a_ref, w1_ref, g1_ref):
    dinvb = dinvb_ref[0]
    pre = (sp_ref[0, 0] + sp_ref[0, 1] + g_ref[0]) * dinvb + b_ref[...]
    z = jnp.where(pre >= 0, pre, a_ref[...] * pre)
    h = jnp.dot(z, w1_ref[...], preferred_element_type=jnp.float32)
    g1_ref[0] = h * dinvb


def _tc_mid(sp, g, dinvb, b0, alpha, w1):
    return pl.pallas_call(
        _mid_body,
        grid=(_T, _N // _BR),
        in_specs=[
            pl.BlockSpec((1, _NC, _BR, _D), lambda t, i: (t, 0, i, 0)),
            pl.BlockSpec((1, _BR, _D), lambda t, i: (t, i, 0)),
            pl.BlockSpec((1, _BR, _D), lambda t, i: (t, i, 0)),
            pl.BlockSpec((1, _D), lambda t, i: (0, 0)),
            pl.BlockSpec((1, _D), lambda t, i: (0, 0)),
            pl.BlockSpec((_D, _D), lambda t, i: (0, 0)),
        ],
        out_specs=pl.BlockSpec((1, _BR, _D), lambda t, i: (t, i, 0)),
        out_shape=jax.ShapeDtypeStruct((_T, _N, _D), jnp.float32),
    )(sp, g, dinvb, b0, alpha, w1)


def _fin_body(sp_ref, g_ref, dinvb_ref, b_ref, a_ref, coeff_ref, out_ref):
    acc = jnp.zeros((_BR, _D), jnp.float32)
    for t in range(_T):
        pre = ((sp_ref[t, 0] + sp_ref[t, 1] + g_ref[t]) * dinvb_ref[t]
               + b_ref[...])
        z = jnp.where(pre >= 0, pre, a_ref[...] * pre)
        acc = acc + coeff_ref[t:t + 1, :] * z
    out_ref[...] = acc


def _tc_fin(sp, g, dinvb, b1, alpha, coeffb):
    return pl.pallas_call(
        _fin_body,
        grid=(_N // _BR,),
        in_specs=[
            pl.BlockSpec((_T, _NC, _BR, _D), lambda i: (0, 0, i, 0)),
            pl.BlockSpec((_T, _BR, _D), lambda i: (0, i, 0)),
            pl.BlockSpec((_T, _BR, _D), lambda i: (0, i, 0)),
            pl.BlockSpec((1, _D), lambda i: (0, 0)),
            pl.BlockSpec((1, _D), lambda i: (0, 0)),
            pl.BlockSpec((_T, _D), lambda i: (0, 0)),
        ],
        out_specs=pl.BlockSpec((_BR, _D), lambda i: (i, 0)),
        out_shape=jax.ShapeDtypeStruct((_N, _D), jnp.float32),
    )(sp, g, dinvb, b1, alpha, coeffb)


def kernel(x, edge_index, edge_weight, coeff, W0, b0, W1, b1, alpha):
    padc = 60
    src = jnp.concatenate(
        [edge_index[0].reshape(_CHUNKS, _K),
         jnp.zeros((padc, _K), jnp.int32)])
    pad_dst = (jnp.arange(padc * _K, dtype=jnp.int32) % _N).reshape(padc, _K)
    dst = jnp.concatenate(
        [edge_index[1].reshape(_CHUNKS, _K), pad_dst])
    w3 = jnp.concatenate(
        [edge_weight.reshape(_T, _CHUNKS, _K),
         jnp.zeros((_T, padc, _K), jnp.float32)], axis=1)
    zero128 = jnp.zeros((_NPAD, _D), jnp.float32)
    ones128 = jnp.ones((_N, _D), jnp.float32)
    b0r = b0.reshape(1, _D)
    b1r = b1.reshape(1, _D)
    ar = alpha.reshape(1, _D)
    coeffb = jnp.broadcast_to(coeff.reshape(_T, 1), (_T, _D))

    degp = jnp.stack([_sc_agg(src, dst, w3[t], ones128, zero128)
                      for t in range(_T)])
    g, dinvb = _tc_prep(degp[:, :, :_N, :], x, W0)

    sp = jnp.stack([_sc_agg(src, dst, w3[t], g[t], zero128)
                    for t in range(_T)])
    g1 = _tc_mid(sp[:, :, :_N, :], g, dinvb, b0r, ar, W1)

    spf = jnp.stack([_sc_agg(src, dst, w3[t], g1[t], zero128)
                     for t in range(_T)])
    return _tc_fin(spf[:, :, :_N, :], g1, dinvb, b1r, ar, coeffb)


# speedup vs baseline: 1.8785x; 1.8785x over previous
import functools

import jax
import jax.numpy as jnp
from jax import lax
from jax.experimental import pallas as pl
from jax.experimental.pallas import tpu as pltpu
from jax.experimental.pallas import tpu_sc as plsc

_N = 10000
_E = 320000
_D = 128
_T = 3
_NC = 2
_NS = 16
_NW = _NC * _NS
_K = 80
_CHUNKS = _E // _K
_CP = 4096
_CPWP = _CP // _NW
_MC = 8
_NMAC = _CPWP // _MC
_NQ = _CPWP // 4
_NPAD = 10112
_RPT = _NPAD // _NS


def _agg_body(src_hbm, dst_hbm, w_hbm, g_hbm, zero_hbm, out_hbm,
              sidx, didx, wv, rows0, rows1, rows2, rows3, acc,
              gsem, ssem, isem):
    c = lax.axis_index("c")
    s = lax.axis_index("s")
    r0 = s * _RPT
    pltpu.sync_copy(zero_hbm.at[pl.ds(r0, _RPT)], acc.at[pl.ds(r0, _RPT)])
    plsc.subcore_barrier()

    wid = s * _NC + c
    base = wid * _CPWP
    rows = (rows0, rows1, rows2, rows3)

    def issue_idx(m, mb):
        off = base + m * _MC
        pltpu.async_copy(src_hbm.at[pl.ds(off, _MC)], sidx.at[mb], isem.at[mb])
        pltpu.async_copy(dst_hbm.at[pl.ds(off, _MC)], didx.at[mb], isem.at[mb])
        pltpu.async_copy(w_hbm.at[pl.ds(off, _MC)], wv.at[mb], isem.at[mb])

    def wait_idx(mb):
        pltpu.make_async_copy(src_hbm.at[pl.ds(0, _MC)], sidx.at[mb],
                              isem.at[mb]).wait()
        pltpu.make_async_copy(dst_hbm.at[pl.ds(0, _MC)], didx.at[mb],
                              isem.at[mb]).wait()
        pltpu.make_async_copy(w_hbm.at[pl.ds(0, _MC)], wv.at[mb],
                              isem.at[mb]).wait()

    def issue_gather(mb, j, b):
        pltpu.async_copy(g_hbm.at[sidx.at[mb, j]], rows[b], gsem.at[b])

    def wait_gather(b):
        pltpu.make_async_copy(g_hbm.at[sidx.at[0, 0]], rows[b],
                              gsem.at[b]).wait()

    def issue_scatter(mb, j, b):
        pltpu.async_copy(rows[b], acc.at[didx.at[mb, j]], ssem.at[b],
                         add=True)

    def wait_scatter(b):
        pltpu.make_async_copy(rows[b], acc.at[didx.at[0, 0]],
                              ssem.at[b]).wait()

    issue_idx(0, 0)
    wait_idx(0)
    issue_gather(0, 0, 0)

    def quad(q, carry):
        m = q // 2
        mb = m % 2
        for i in range(4):
            wait_gather(i)
            nxt = (i + 1) % 4
            if i < 3:
                @pl.when(q > 0)
                def _():
                    wait_scatter(nxt)
                issue_gather(mb, (q % 2) * 4 + i + 1, nxt)
            else:
                wait_scatter(0)
                @pl.when(jnp.logical_and(q % 2 == 0, m + 1 < _NMAC))
                def _():
                    issue_idx(m + 1, 1 - mb)

                @pl.when(jnp.logical_and(q % 2 == 1, q < _NQ - 1))
                def _():
                    wait_idx(1 - mb)

                @pl.when(q < _NQ - 1)
                def _():
                    nmb = jnp.where(q % 2 == 1, 1 - mb, mb)
                    nj = jnp.where(q % 2 == 1, 0, 4)
                    issue_gather(nmb, nj, 0)
            jj = (q % 2) * 4 + i

            def grp(jb, carry2, _i=i, _jj=jj):
                w16v = wv[mb, _jj, pl.ds(jb * 16, 16)]
                for l in range(16):
                    w16 = lax.broadcast(w16v[l], (16,))
                    e = jb * 16 + l
                    for cb in range(8):
                        v = rows[_i][e, pl.ds(cb * 16, 16)]
                        rows[_i][e, pl.ds(cb * 16, 16)] = v * w16
                return carry2
            lax.fori_loop(0, _K // 16, grp, 0)
            issue_scatter(mb, jj, i)
        return carry
    lax.fori_loop(0, _NQ, quad, 0)
    for b in (1, 2, 3):
        wait_scatter(b)
    plsc.subcore_barrier()
    pltpu.sync_copy(acc.at[pl.ds(r0, _RPT)], out_hbm.at[c, pl.ds(r0, _RPT)])


_sc_agg = functools.partial(
    pl.kernel,
    out_type=jax.ShapeDtypeStruct((_NC, _NPAD, _D), jnp.float32),
    mesh=plsc.VectorSubcoreMesh(core_axis_name="c", subcore_axis_name="s"),
    scratch_types=[
        pltpu.VMEM((2, _MC, _K), jnp.int32),
        pltpu.VMEM((2, _MC, _K), jnp.int32),
        pltpu.VMEM((2, _MC, _K), jnp.float32),
        pltpu.VMEM((_K, _D), jnp.float32),
        pltpu.VMEM((_K, _D), jnp.float32),
        pltpu.VMEM((_K, _D), jnp.float32),
        pltpu.VMEM((_K, _D), jnp.float32),
        pltpu.VMEM_SHARED((_NPAD, _D), jnp.float32),
        pltpu.SemaphoreType.DMA((4,)),
        pltpu.SemaphoreType.DMA((4,)),
        pltpu.SemaphoreType.DMA((2,)),
    ],
)(_agg_body)


_BR = 1000


def _prep_body(degp_ref, x_ref, w0_ref, g_ref, dinvb_ref):
    d = degp_ref[0, 0] + degp_ref[0, 1]
    dinvb = lax.rsqrt(d + 1.0)
    hw = jnp.dot(x_ref[...], w0_ref[...], preferred_element_type=jnp.float32)
    g_ref[0] = hw * dinvb
    dinvb_ref[0] = dinvb


def _tc_prep(degp, x, w0):
    return pl.pallas_call(
        _prep_body,
        grid=(_T, _N // _BR),
        in_specs=[
            pl.BlockSpec((1, _NC, _BR, _D), lambda t, i: (t, 0, i, 0)),
            pl.BlockSpec((_BR, _D), lambda t, i: (i, 0)),
            pl.BlockSpec((_D, _D), lambda t, i: (0, 0)),
        ],
        out_specs=[
            pl.BlockSpec((1, _BR, _D), lambda t, i: (t, i, 0)),
            pl.BlockSpec((1, _BR, _D), lambda t, i: (t, i, 0)),
        ],
        out_shape=[
            jax.ShapeDtypeStruct((_T, _N, _D), jnp.float32),
            jax.ShapeDtypeStruct((_T, _N, _D), jnp.float32),
        ],
    )(degp, x, w0)


def _mid_body(sp_ref, g_ref, dinvb_ref, b_ref, a_ref, w1_ref, g1_ref):
    dinvb = dinvb_ref[0]
    pre = (sp_ref[0, 0] + sp_ref[0, 1] + g_ref[0]) * dinvb + b_ref[...]
    z = jnp.where(pre >= 0, pre, a_ref[...] * pre)
    h = jnp.dot(z, w1_ref[...], preferred_element_type=jnp.float32)
    g1_ref[0] = h * dinvb


def _tc_mid(sp, g, dinvb, b0, alpha, w1):
    return pl.pallas_call(
        _mid_body,
        grid=(_T, _N // _BR),
        in_specs=[
            pl.BlockSpec((1, _NC, _BR, _D), lambda t, i: (t, 0, i, 0)),
            pl.BlockSpec((1, _BR, _D), lambda t, i: (t, i, 0)),
            pl.BlockSpec((1, _BR, _D), lambda t, i: (t, i, 0)),
            pl.BlockSpec((1, _D), lambda t, i: (0, 0)),
            pl.BlockSpec((1, _D), lambda t, i: (0, 0)),
            pl.BlockSpec((_D, _D), lambda t, i: (0, 0)),
        ],
        out_specs=pl.BlockSpec((1, _BR, _D), lambda t, i: (t, i, 0)),
        out_shape=jax.ShapeDtypeStruct((_T, _N, _D), jnp.float32),
    )(sp, g, dinvb, b0, alpha, w1)


def _fin_body(sp_ref, g_ref, dinvb_ref, b_ref, a_ref, coeff_ref, out_ref):
    acc = jnp.zeros((_BR, _D), jnp.float32)
    for t in range(_T):
        pre = ((sp_ref[t, 0] + sp_ref[t, 1] + g_ref[t]) * dinvb_ref[t]
               + b_ref[...])
        z = jnp.where(pre >= 0, pre, a_ref[...] * pre)
        acc = acc + coeff_ref[t:t + 1, :] * z
    out_ref[...] = acc


def _tc_fin(sp, g, dinvb, b1, alpha, coeffb):
    return pl.pallas_call(
        _fin_body,
        grid=(_N // _BR,),
        in_specs=[
            pl.BlockSpec((_T, _NC, _BR, _D), lambda i: (0, 0, i, 0)),
            pl.BlockSpec((_T, _BR, _D), lambda i: (0, i, 0)),
            pl.BlockSpec((_T, _BR, _D), lambda i: (0, i, 0)),
            pl.BlockSpec((1, _D), lambda i: (0, 0)),
            pl.BlockSpec((1, _D), lambda i: (0, 0)),
            pl.BlockSpec((_T, _D), lambda i: (0, 0)),
        ],
        out_specs=pl.BlockSpec((_BR, _D), lambda i: (i, 0)),
        out_shape=jax.ShapeDtypeStruct((_N, _D), jnp.float32),
    )(sp, g, dinvb, b1, alpha, coeffb)


def kernel(x, edge_index, edge_weight, coeff, W0, b0, W1, b1, alpha):
    padc = _CP - _CHUNKS
    pad_idx = (jnp.arange(padc * _K, dtype=jnp.int32) % _N).reshape(padc, _K)
    src = jnp.concatenate([edge_index[0].reshape(_CHUNKS, _K), pad_idx])
    dst = jnp.concatenate([edge_index[1].reshape(_CHUNKS, _K), pad_idx])
    w3 = jnp.concatenate(
        [edge_weight.reshape(_T, _CHUNKS, _K),
         jnp.zeros((_T, padc, _K), jnp.float32)], axis=1)
    zero128 = jnp.zeros((_NPAD, _D), jnp.float32)
    ones128 = jnp.ones((_N, _D), jnp.float32)
    b0r = b0.reshape(1, _D)
    b1r = b1.reshape(1, _D)
    ar = alpha.reshape(1, _D)
    coeffb = jnp.broadcast_to(coeff.reshape(_T, 1), (_T, _D))

    degp = jnp.stack([_sc_agg(src, dst, w3[t], ones128, zero128)
                      for t in range(_T)])
    g, dinvb = _tc_prep(degp[:, :, :_N, :], x, W0)

    sp = jnp.stack([_sc_agg(src, dst, w3[t], g[t], zero128)
                    for t in range(_T)])
    g1 = _tc_mid(sp[:, :, :_N, :], g, dinvb, b0r, ar, W1)

    spf = jnp.stack([_sc_agg(src, dst, w3[t], g1[t], zero128)
                     for t in range(_T)])
    return _tc_fin(spf[:, :, :_N, :], g1, dinvb, b1r, ar, coeffb)


# speedup vs baseline: 2.5521x; 1.3586x over previous
import functools

import jax
import jax.numpy as jnp
from jax import lax
from jax.experimental import pallas as pl
from jax.experimental.pallas import tpu as pltpu
from jax.experimental.pallas import tpu_sc as plsc

_N = 10000
_E = 320000
_D = 128
_T = 3
_NC = 2
_NS = 16
_NW = _NC * _NS
_K = 80
_CHUNKS = _E // _K
_CP = 4096
_CPWP = _CP // _NW
_MC = 8
_NMAC = _CPWP // _MC
_NQ = _CPWP // 4
_NPAD = 10112
_RPT = _NPAD // _NS


def _agg_body(src_hbm, dst_hbm, w_hbm, g_hbm, zero_hbm, out_hbm,
              sidx, didx, wv, rows0, rows1, rows2, rows3, acc,
              gsem, ssem, isem):
    c = lax.axis_index("c")
    s = lax.axis_index("s")
    r0 = s * _RPT
    pltpu.sync_copy(zero_hbm.at[pl.ds(r0, _RPT)], acc.at[pl.ds(r0, _RPT)])
    plsc.subcore_barrier()

    wid = s * _NC + c
    base = wid * _CPWP
    rows = (rows0, rows1, rows2, rows3)

    def issue_idx(m, mb):
        off = base + m * _MC
        pltpu.async_copy(src_hbm.at[pl.ds(off, _MC)], sidx.at[mb], isem.at[mb])
        pltpu.async_copy(dst_hbm.at[pl.ds(off, _MC)], didx.at[mb], isem.at[mb])
        pltpu.async_copy(w_hbm.at[pl.ds(off, _MC)], wv.at[mb], isem.at[mb])

    def wait_idx(mb):
        pltpu.make_async_copy(src_hbm.at[pl.ds(0, _MC)], sidx.at[mb],
                              isem.at[mb]).wait()
        pltpu.make_async_copy(dst_hbm.at[pl.ds(0, _MC)], didx.at[mb],
                              isem.at[mb]).wait()
        pltpu.make_async_copy(w_hbm.at[pl.ds(0, _MC)], wv.at[mb],
                              isem.at[mb]).wait()

    def issue_gather(mb, j, b):
        pltpu.async_copy(g_hbm.at[sidx.at[mb, j]], rows[b], gsem.at[b])

    def wait_gather(b):
        pltpu.make_async_copy(g_hbm.at[sidx.at[0, 0]], rows[b],
                              gsem.at[b]).wait()

    def issue_scatter(mb, j, b):
        pltpu.async_copy(rows[b], acc.at[didx.at[mb, j]], ssem.at[b],
                         add=True)

    def wait_scatter(b):
        pltpu.make_async_copy(rows[b], acc.at[didx.at[0, 0]],
                              ssem.at[b]).wait()

    issue_idx(0, 0)
    wait_idx(0)
    issue_gather(0, 0, 0)

    def quad(q, carry):
        m = q // 2
        mb = m % 2
        for i in range(4):
            wait_gather(i)
            nxt = (i + 1) % 4
            if i < 3:
                @pl.when(q > 0)
                def _():
                    wait_scatter(nxt)
                issue_gather(mb, (q % 2) * 4 + i + 1, nxt)
            else:
                wait_scatter(0)
                @pl.when(jnp.logical_and(q % 2 == 0, m + 1 < _NMAC))
                def _():
                    issue_idx(m + 1, 1 - mb)

                @pl.when(jnp.logical_and(q % 2 == 1, q < _NQ - 1))
                def _():
                    wait_idx(1 - mb)

                @pl.when(q < _NQ - 1)
                def _():
                    nmb = jnp.where(q % 2 == 1, 1 - mb, mb)
                    nj = jnp.where(q % 2 == 1, 0, 4)
                    issue_gather(nmb, nj, 0)
            jj = (q % 2) * 4 + i

            def grp(jb, carry2, _i=i, _jj=jj):
                w16v = wv[mb, _jj, pl.ds(jb * 16, 16)]
                for l in range(16):
                    w16 = lax.broadcast(w16v[l], (16,))
                    e = jb * 16 + l
                    for cb in range(8):
                        v = rows[_i][e, pl.ds(cb * 16, 16)]
                        rows[_i][e, pl.ds(cb * 16, 16)] = v * w16
                return carry2
            lax.fori_loop(0, _K // 16, grp, 0)
            issue_scatter(mb, jj, i)
        return carry
    lax.fori_loop(0, _NQ, quad, 0)
    for b in (1, 2, 3):
        wait_scatter(b)
    plsc.subcore_barrier()
    pltpu.sync_copy(acc.at[pl.ds(r0, _RPT)], out_hbm.at[c, pl.ds(r0, _RPT)])


_sc_agg = functools.partial(
    pl.kernel,
    out_type=jax.ShapeDtypeStruct((_NC, _NPAD, _D), jnp.float32),
    mesh=plsc.VectorSubcoreMesh(core_axis_name="c", subcore_axis_name="s"),
    scratch_types=[
        pltpu.VMEM((2, _MC, _K), jnp.int32),
        pltpu.VMEM((2, _MC, _K), jnp.int32),
        pltpu.VMEM((2, _MC, _K), jnp.float32),
        pltpu.VMEM((_K, _D), jnp.float32),
        pltpu.VMEM((_K, _D), jnp.float32),
        pltpu.VMEM((_K, _D), jnp.float32),
        pltpu.VMEM((_K, _D), jnp.float32),
        pltpu.VMEM_SHARED((_NPAD, _D), jnp.float32),
        pltpu.SemaphoreType.DMA((4,)),
        pltpu.SemaphoreType.DMA((4,)),
        pltpu.SemaphoreType.DMA((2,)),
    ],
)(_agg_body)


def _deg_body(dst_hbm, w_hbm, zero_hbm, out_hbm,
              didx, wv, rows0, rows1, rows2, rows3, acc, ssem, isem):
    c = lax.axis_index("c")
    s = lax.axis_index("s")
    r0 = s * _RPT
    pltpu.sync_copy(zero_hbm.at[pl.ds(r0, _RPT)], acc.at[pl.ds(r0, _RPT)])
    rows = (rows0, rows1, rows2, rows3)
    z16 = jnp.zeros((16,), jnp.float32)

    def zrow(e, carry):
        for b in range(4):
            for cb in range(_T, 8):
                rows[b][e, pl.ds(cb * 16, 16)] = z16
        return carry
    lax.fori_loop(0, _K, zrow, 0)
    plsc.subcore_barrier()

    wid = s * _NC + c
    base = wid * _CPWP

    def issue_idx(m, mb):
        off = base + m * _MC
        pltpu.async_copy(dst_hbm.at[pl.ds(off, _MC)], didx.at[mb], isem.at[mb])
        for t in range(_T):
            pltpu.async_copy(w_hbm.at[t, pl.ds(off, _MC)], wv.at[mb, t],
                             isem.at[mb])

    def wait_idx(mb):
        pltpu.make_async_copy(dst_hbm.at[pl.ds(0, _MC)], didx.at[mb],
                              isem.at[mb]).wait()
        for t in range(_T):
            pltpu.make_async_copy(w_hbm.at[t, pl.ds(0, _MC)], wv.at[mb, t],
                                  isem.at[mb]).wait()

    def issue_scatter(mb, j, b):
        pltpu.async_copy(rows[b], acc.at[didx.at[mb, j]], ssem.at[b],
                         add=True)

    def wait_scatter(b):
        pltpu.make_async_copy(rows[b], acc.at[didx.at[0, 0]],
                              ssem.at[b]).wait()

    issue_idx(0, 0)
    wait_idx(0)

    def quad(q, carry):
        m = q // 2
        mb = m % 2
        for i in range(4):
            @pl.when(q > 0)
            def _():
                wait_scatter(i)
            jj = (q % 2) * 4 + i

            def grp(jb, carry2, _i=i, _jj=jj):
                for t in range(_T):
                    w16v = wv[mb, t, _jj, pl.ds(jb * 16, 16)]
                    for l in range(16):
                        w16 = lax.broadcast(w16v[l], (16,))
                        rows[_i][jb * 16 + l, pl.ds(16 * t, 16)] = w16
                return carry2
            lax.fori_loop(0, _K // 16, grp, 0)
            issue_scatter(mb, jj, i)
            if i == 3:
                @pl.when(jnp.logical_and(q % 2 == 0, m + 1 < _NMAC))
                def _():
                    issue_idx(m + 1, 1 - mb)

                @pl.when(jnp.logical_and(q % 2 == 1, q < _NQ - 1))
                def _():
                    wait_idx(1 - mb)
        return carry
    lax.fori_loop(0, _NQ, quad, 0)
    for b in range(4):
        wait_scatter(b)
    plsc.subcore_barrier()
    pltpu.sync_copy(acc.at[pl.ds(r0, _RPT)], out_hbm.at[c, pl.ds(r0, _RPT)])


_sc_deg = functools.partial(
    pl.kernel,
    out_type=jax.ShapeDtypeStruct((_NC, _NPAD, _D), jnp.float32),
    mesh=plsc.VectorSubcoreMesh(core_axis_name="c", subcore_axis_name="s"),
    scratch_types=[
        pltpu.VMEM((2, _MC, _K), jnp.int32),
        pltpu.VMEM((2, _T, _MC, _K), jnp.float32),
        pltpu.VMEM((_K, _D), jnp.float32),
        pltpu.VMEM((_K, _D), jnp.float32),
        pltpu.VMEM((_K, _D), jnp.float32),
        pltpu.VMEM((_K, _D), jnp.float32),
        pltpu.VMEM_SHARED((_NPAD, _D), jnp.float32),
        pltpu.SemaphoreType.DMA((4,)),
        pltpu.SemaphoreType.DMA((2,)),
    ],
)(_deg_body)


_BR = 1000


def _prep_body(degp_ref, x_ref, w0_ref, g_ref, dinvb_ref):
    d = degp_ref[0] + degp_ref[1]
    hw = jnp.dot(x_ref[...], w0_ref[...], preferred_element_type=jnp.float32)
    for t in range(_T):
        deg = d[:, 16 * t:16 * t + 1] + 1.0
        dinvb = jnp.broadcast_to(lax.rsqrt(deg), (_BR, _D))
        g_ref[t] = hw * dinvb
        dinvb_ref[t] = dinvb


def _tc_prep(degp, x, w0):
    return pl.pallas_call(
        _prep_body,
        grid=(_N // _BR,),
        in_specs=[
            pl.BlockSpec((_NC, _BR, _D), lambda i: (0, i, 0)),
            pl.BlockSpec((_BR, _D), lambda i: (i, 0)),
            pl.BlockSpec((_D, _D), lambda i: (0, 0)),
        ],
        out_specs=[
            pl.BlockSpec((_T, _BR, _D), lambda i: (0, i, 0)),
            pl.BlockSpec((_T, _BR, _D), lambda i: (0, i, 0)),
        ],
        out_shape=[
            jax.ShapeDtypeStruct((_T, _N, _D), jnp.float32),
            jax.ShapeDtypeStruct((_T, _N, _D), jnp.float32),
        ],
    )(degp, x, w0)


def _mid_body(sp_ref, g_ref, dinvb_ref, b_ref, a_ref, w1_ref, g1_ref):
    dinvb = dinvb_ref[0]
    pre = (sp_ref[0, 0] + sp_ref[0, 1] + g_ref[0]) * dinvb + b_ref[...]
    z = jnp.where(pre >= 0, pre, a_ref[...] * pre)
    h = jnp.dot(z, w1_ref[...], preferred_element_type=jnp.float32)
    g1_ref[0] = h * dinvb


def _tc_mid(sp, g, dinvb, b0, alpha, w1):
    return pl.pallas_call(
        _mid_body,
        grid=(_T, _N // _BR),
        in_specs=[
            pl.BlockSpec((1, _NC, _BR, _D), lambda t, i: (t, 0, i, 0)),
            pl.BlockSpec((1, _BR, _D), lambda t, i: (t, i, 0)),
            pl.BlockSpec((1, _BR, _D), lambda t, i: (t, i, 0)),
            pl.BlockSpec((1, _D), lambda t, i: (0, 0)),
            pl.BlockSpec((1, _D), lambda t, i: (0, 0)),
            pl.BlockSpec((_D, _D), lambda t, i: (0, 0)),
        ],
        out_specs=pl.BlockSpec((1, _BR, _D), lambda t, i: (t, i, 0)),
        out_shape=jax.ShapeDtypeStruct((_T, _N, _D), jnp.float32),
    )(sp, g, dinvb, b0, alpha, w1)


def _fin_body(sp_ref, g_ref, dinvb_ref, b_ref, a_ref, coeff_ref, out_ref):
    acc = jnp.zeros((_BR, _D), jnp.float32)
    for t in range(_T):
        pre = ((sp_ref[t, 0] + sp_ref[t, 1] + g_ref[t]) * dinvb_ref[t]
               + b_ref[...])
        z = jnp.where(pre >= 0, pre, a_ref[...] * pre)
        acc = acc + coeff_ref[t:t + 1, :] * z
    out_ref[...] = acc


def _tc_fin(sp, g, dinvb, b1, alpha, coeffb):
    return pl.pallas_call(
        _fin_body,
        grid=(_N // _BR,),
        in_specs=[
            pl.BlockSpec((_T, _NC, _BR, _D), lambda i: (0, 0, i, 0)),
            pl.BlockSpec((_T, _BR, _D), lambda i: (0, i, 0)),
            pl.BlockSpec((_T, _BR, _D), lambda i: (0, i, 0)),
            pl.BlockSpec((1, _D), lambda i: (0, 0)),
            pl.BlockSpec((1, _D), lambda i: (0, 0)),
            pl.BlockSpec((_T, _D), lambda i: (0, 0)),
        ],
        out_specs=pl.BlockSpec((_BR, _D), lambda i: (i, 0)),
        out_shape=jax.ShapeDtypeStruct((_N, _D), jnp.float32),
    )(sp, g, dinvb, b1, alpha, coeffb)


def kernel(x, edge_index, edge_weight, coeff, W0, b0, W1, b1, alpha):
    padc = _CP - _CHUNKS
    pad_idx = (jnp.arange(padc * _K, dtype=jnp.int32) % _N).reshape(padc, _K)
    src = jnp.concatenate([edge_index[0].reshape(_CHUNKS, _K), pad_idx])
    dst = jnp.concatenate([edge_index[1].reshape(_CHUNKS, _K), pad_idx])
    w3 = jnp.concatenate(
        [edge_weight.reshape(_T, _CHUNKS, _K),
         jnp.zeros((_T, padc, _K), jnp.float32)], axis=1)
    zero128 = jnp.zeros((_NPAD, _D), jnp.float32)
    b0r = b0.reshape(1, _D)
    b1r = b1.reshape(1, _D)
    ar = alpha.reshape(1, _D)
    coeffb = jnp.broadcast_to(coeff.reshape(_T, 1), (_T, _D))

    degp = _sc_deg(dst, w3, zero128)
    g, dinvb = _tc_prep(degp, x, W0)

    sp = jnp.stack([_sc_agg(src, dst, w3[t], g[t], zero128)
                    for t in range(_T)])
    g1 = _tc_mid(sp[:, :, :_N, :], g, dinvb, b0r, ar, W1)

    spf = jnp.stack([_sc_agg(src, dst, w3[t], g1[t], zero128)
                     for t in range(_T)])
    return _tc_fin(spf[:, :, :_N, :], g1, dinvb, b1r, ar, coeffb)
